# Initial kernel scaffold; baseline (speedup 1.0000x reference)
import jax
import jax.numpy as jnp
from jax.experimental import pallas as pl


def kernel(x, codebook):
    raise NotImplementedError("write your pallas kernel here")


# speedup vs baseline: 1.0140x; 1.0140x over previous
import functools

import jax
import jax.numpy as jnp
from jax.experimental import pallas as pl

_BETA = 0.25


def _vq_body(xt_ref, xr_ref, cb_ref, xn_ref, cn_ref, xq_ref, idx_ref, acc_ref):
    i = pl.program_id(0)
    cb = cb_ref[...]
    xt = xt_ref[...]
    xr = xr_ref[...]
    xnorm = xn_ref[...]
    cnorm = cn_ref[...]
    mm = jnp.dot(xt, cb.T, preferred_element_type=jnp.float32)
    d = (xnorm + cnorm) - 2.0 * mm
    m = jnp.min(d, axis=1, keepdims=True)
    ks = jax.lax.broadcasted_iota(jnp.int32, d.shape, 1)
    idx = jnp.min(jnp.where(d == m, ks, d.shape[1]), axis=1).astype(jnp.int32)
    onehot = (ks == idx[:, None]).astype(jnp.float32)
    xq = jnp.dot(onehot, cb, preferred_element_type=jnp.float32,
                 precision=jax.lax.Precision.HIGHEST)
    diff = xq - xr
    xq_ref[...] = xr + diff
    idx_ref[...] = idx.reshape(1, 1, -1)
    part = jnp.sum(diff * diff).reshape(1, 1)

    @pl.when(i == 0)
    def _():
        acc_ref[...] = part

    @pl.when(i != 0)
    def _():
        acc_ref[...] = acc_ref[...] + part


@functools.partial(jax.jit, static_argnames=("interpret",))
def kernel(x, codebook, interpret=False):
    B, C, H, W = x.shape
    N = B * H * W
    K, D = codebook.shape
    BLK = 512
    grid = N // BLK

    x_t = jnp.transpose(x, (0, 2, 3, 1)).reshape(N, C)
    x_r = x.reshape(N, C)
    xn = jnp.sum(x_t ** 2, axis=1, keepdims=True)
    cn = jnp.sum(codebook ** 2, axis=1).reshape(1, K)

    xq2d, idx3d, acc = pl.pallas_call(
        _vq_body,
        grid=(grid,),
        in_specs=[
            pl.BlockSpec((BLK, C), lambda i: (i, 0)),
            pl.BlockSpec((BLK, C), lambda i: (i, 0)),
            pl.BlockSpec((K, D), lambda i: (0, 0)),
            pl.BlockSpec((BLK, 1), lambda i: (i, 0)),
            pl.BlockSpec((1, K), lambda i: (0, 0)),
        ],
        out_specs=[
            pl.BlockSpec((BLK, C), lambda i: (i, 0)),
            pl.BlockSpec((1, 1, BLK), lambda i: (i, 0, 0)),
            pl.BlockSpec((1, 1), lambda i: (0, 0)),
        ],
        out_shape=[
            jax.ShapeDtypeStruct((N, C), jnp.float32),
            jax.ShapeDtypeStruct((grid, 1, BLK), jnp.int32),
            jax.ShapeDtypeStruct((1, 1), jnp.float32),
        ],
        interpret=interpret,
    )(x_t, x_r, codebook, xn, cn)

    x_q = xq2d.reshape(x.shape)
    indices = idx3d.reshape(N)
    m = acc[0, 0] / jnp.float32(N * C)
    loss = m + _BETA * m
    return (x_q, indices, loss)


# speedup vs baseline: 1.1440x; 1.1283x over previous
import functools

import jax
import jax.numpy as jnp
from jax.experimental import pallas as pl

_BETA = 0.25


def _vq_body(xt_ref, xr_ref, cb_ref, ch_ref, cm_ref, cl_ref, xn_ref, cn_ref,
             xq_ref, idx_ref, acc_ref):
    i = pl.program_id(0)
    cb = cb_ref[...]
    xt = xt_ref[...]
    xr = xr_ref[...]
    xnorm = xn_ref[...]
    cnorm = cn_ref[...]
    mm = jnp.dot(xt, cb.T, preferred_element_type=jnp.float32)
    d = (xnorm + cnorm) - 2.0 * mm
    m = jnp.min(d, axis=1, keepdims=True)
    ks = jax.lax.broadcasted_iota(jnp.int32, d.shape, 1)
    idx = jnp.min(jnp.where(d == m, ks, d.shape[1]), axis=1).astype(jnp.int32)
    onehot = (ks == idx[:, None]).astype(jnp.float32)
    q_hi = jnp.dot(onehot, ch_ref[...], preferred_element_type=jnp.float32)
    q_mid = jnp.dot(onehot, cm_ref[...], preferred_element_type=jnp.float32)
    q_lo = jnp.dot(onehot, cl_ref[...], preferred_element_type=jnp.float32)
    xq = (q_hi + q_mid) + q_lo
    diff = xq - xr
    xq_ref[...] = xr + diff
    idx_ref[...] = idx.reshape(1, 1, -1)
    part = jnp.sum(diff * diff).reshape(1, 1)

    @pl.when(i == 0)
    def _():
        acc_ref[...] = part

    @pl.when(i != 0)
    def _():
        acc_ref[...] = acc_ref[...] + part


@functools.partial(jax.jit, static_argnames=("interpret",))
def kernel(x, codebook, interpret=False):
    B, C, H, W = x.shape
    N = B * H * W
    K, D = codebook.shape
    BLK = 512
    grid = N // BLK

    x_t = jnp.transpose(x, (0, 2, 3, 1)).reshape(N, C)
    x_r = x.reshape(N, C)
    xn = jnp.sum(x_t ** 2, axis=1, keepdims=True)
    cn = jnp.sum(codebook ** 2, axis=1).reshape(1, K)
    cb_hi = codebook.astype(jnp.bfloat16).astype(jnp.float32)
    r1 = codebook - cb_hi
    cb_mid = r1.astype(jnp.bfloat16).astype(jnp.float32)
    cb_lo = r1 - cb_mid

    xq2d, idx3d, acc = pl.pallas_call(
        _vq_body,
        grid=(grid,),
        in_specs=[
            pl.BlockSpec((BLK, C), lambda i: (i, 0)),
            pl.BlockSpec((BLK, C), lambda i: (i, 0)),
            pl.BlockSpec((K, D), lambda i: (0, 0)),
            pl.BlockSpec((K, D), lambda i: (0, 0)),
            pl.BlockSpec((K, D), lambda i: (0, 0)),
            pl.BlockSpec((K, D), lambda i: (0, 0)),
            pl.BlockSpec((BLK, 1), lambda i: (i, 0)),
            pl.BlockSpec((1, K), lambda i: (0, 0)),
        ],
        out_specs=[
            pl.BlockSpec((BLK, C), lambda i: (i, 0)),
            pl.BlockSpec((1, 1, BLK), lambda i: (i, 0, 0)),
            pl.BlockSpec((1, 1), lambda i: (0, 0)),
        ],
        out_shape=[
            jax.ShapeDtypeStruct((N, C), jnp.float32),
            jax.ShapeDtypeStruct((grid, 1, BLK), jnp.int32),
            jax.ShapeDtypeStruct((1, 1), jnp.float32),
        ],
        interpret=interpret,
    )(x_t, x_r, codebook, cb_hi, cb_mid, cb_lo, xn, cn)

    x_q = xq2d.reshape(x.shape)
    indices = idx3d.reshape(N)
    m = acc[0, 0] / jnp.float32(N * C)
    loss = m + _BETA * m
    return (x_q, indices, loss)
